# Initial kernel scaffold; baseline (speedup 1.0000x reference)
import jax
import jax.numpy as jnp
from jax.experimental import pallas as pl


def kernel(input, coeffs, G_rows, G_cols, G_vals, L_rows, L_cols, L_vals, F_rows, F_cols, F_vals, NS, EW):
    raise NotImplementedError("write your pallas kernel here")


# speedup vs baseline: 9.1433x; 9.1433x over previous
import functools
import jax
import jax.numpy as jnp
from jax import lax
from jax.experimental import pallas as pl
from jax.experimental.pallas import tpu as pltpu

NV = 40962
NV_PREV = 10242
NF = 81920
C = 128
BS = 2
NVP = 41472
TV = 512


def _final_matmul(xT, lap, gv, C0, C1, C23):
    def body(x_ref, l_ref, g_ref, c0_ref, c1_ref, c23_ref, o_ref):
        a = lax.dot_general(c0_ref[...], x_ref[0],
                            (((0,), (1,)), ((), ())),
                            preferred_element_type=jnp.float32)
        b = lax.dot_general(c1_ref[...], l_ref[0],
                            (((0,), (1,)), ((), ())),
                            preferred_element_type=jnp.float32)
        c = lax.dot_general(c23_ref[...], g_ref[0],
                            (((0,), (1,)), ((), ())),
                            preferred_element_type=jnp.float32)
        o_ref[0] = a + b + c

    return pl.pallas_call(
        body,
        grid=(BS, NVP // TV),
        in_specs=[
            pl.BlockSpec((1, TV, 128), lambda b, i: (b, i, 0)),
            pl.BlockSpec((1, TV, 128), lambda b, i: (b, i, 0)),
            pl.BlockSpec((1, TV, 256), lambda b, i: (b, i, 0)),
            pl.BlockSpec((128, 128), lambda b, i: (0, 0)),
            pl.BlockSpec((128, 128), lambda b, i: (0, 0)),
            pl.BlockSpec((256, 128), lambda b, i: (0, 0)),
        ],
        out_specs=pl.BlockSpec((1, 128, TV), lambda b, i: (b, 0, i)),
        out_shape=jax.ShapeDtypeStruct((BS, 128, NVP), jnp.float32),
    )(xT, lap, gv, C0, C1, C23)


def kernel(input, coeffs, G_rows, G_cols, G_vals, L_rows, L_cols, L_vals,
           F_rows, F_cols, F_vals, NS, EW):
    xT = jnp.concatenate(
        [input.transpose(0, 2, 1),
         jnp.ones((BS, NV - NV_PREV, C), jnp.float32)], axis=1)

    gcols = G_cols.reshape(3, NF, 3).transpose(1, 0, 2).reshape(NF, 9)
    gvals = G_vals.reshape(3, NF, 3).transpose(1, 0, 2).reshape(NF, 9)
    lcols = L_cols.reshape(NV, 7)
    lvals = L_vals.reshape(NV, 7)
    fcols = F_cols.reshape(NV, 6)
    fvals = F_vals.reshape(NV, 6)

    xg = xT[:, gcols.reshape(-1), :].reshape(BS, NF, 9, C)
    p = (xg * gvals[None, :, :, None]).reshape(BS, NF, 3, 3, C).sum(3)
    ew_f = (p * EW.T[None, :, :, None].transpose(0, 2, 1, 3)).sum(2)
    ns_f = (p * NS.T[None, :, :, None].transpose(0, 2, 1, 3)).sum(2)
    ewns = jnp.concatenate([ew_f, ns_f], axis=-1)

    xl = xT[:, lcols.reshape(-1), :].reshape(BS, NV, 7, C)
    lap = (xl * lvals[None, :, :, None]).sum(2)

    gvg = ewns[:, fcols.reshape(-1), :].reshape(BS, NV, 6, 256)
    gv = (gvg * fvals[None, :, :, None]).sum(2)

    C4 = coeffs.reshape(C, 4, 128)
    C0 = C4[:, 0, :]
    C1 = C4[:, 1, :]
    C23 = jnp.concatenate([C4[:, 2, :], C4[:, 3, :]], axis=0)

    pad = ((0, 0), (0, NVP - NV), (0, 0))
    out = _final_matmul(jnp.pad(xT, pad), jnp.pad(lap, pad),
                        jnp.pad(gv, pad), C0, C1, C23)
    return out[:, :, :NV]


# speedup vs baseline: 16.9067x; 1.8491x over previous
import functools
import jax
import jax.numpy as jnp
from jax import lax
from jax.experimental import pallas as pl
from jax.experimental.pallas import tpu as pltpu
from jax.experimental.pallas import tpu_sc as plsc

NV = 40962
NV_PREV = 10242
NF = 81920
C = 128
BS = 2
NVP = 41472
TV = 512

NW = 32
CH_A = 64
CH_L = 64
CH_F = 32
TA = BS * NF // CH_A
NCL = NVP // CH_L
TL = BS * NCL
TF = BS * NVP // CH_F


def _sc_mesh():
    return plsc.VectorSubcoreMesh(core_axis_name="c", subcore_axis_name="s")


def _k1(xT, idxg, wg, idxl, wl):
    @functools.partial(
        pl.kernel, mesh=_sc_mesh(),
        out_type=[
            jax.ShapeDtypeStruct((BS * NF, 2 * C), jnp.float32),
            jax.ShapeDtypeStruct((BS * NVP, C), jnp.float32),
        ],
        scratch_types=[
            pltpu.VMEM((9, CH_A), jnp.int32),
            pltpu.VMEM((15, CH_A), jnp.float32),
            pltpu.VMEM((9, CH_A, C), jnp.float32),
            pltpu.VMEM((CH_A, 2 * C), jnp.float32),
            pltpu.VMEM((CH_L, C), jnp.float32),
            pltpu.SemaphoreType.DMA,
        ],
        compiler_params=pltpu.CompilerParams(needs_layout_passes=False),
    )
    def body(xT_h, idxg_h, wg_h, idxl_h, wl_h, ewns_h, lap_h,
             idx_v, w_v, rows_v, outA, outL, sem):
        wid = lax.axis_index("s") * 2 + lax.axis_index("c")

        def a_task(i, _):
            t = wid + i * NW
            pltpu.sync_copy(idxg_h.at[t], idx_v)
            pltpu.sync_copy(wg_h.at[t], w_v)
            cps = [pltpu.async_copy(xT_h.at[idx_v.at[j]], rows_v.at[j], sem)
                   for j in range(9)]
            for cp in cps:
                cp.wait()

            for f0 in range(0, CH_A, 16):
                w = [w_v[m, pl.ds(f0, 16)] for m in range(15)]
                fvec = f0 + lax.iota(jnp.int32, 16)
                jc = [jnp.full((16,), j, jnp.int32) for j in range(9)]

                def chan(ch, _):
                    chv = jnp.full((16,), ch, jnp.int32)
                    v = [plsc.load_gather(rows_v, [jc[j], fvec, chv])
                         for j in range(9)]
                    p0 = v[0] * w[0] + v[1] * w[1] + v[2] * w[2]
                    p1 = v[3] * w[3] + v[4] * w[4] + v[5] * w[5]
                    p2 = v[6] * w[6] + v[7] * w[7] + v[8] * w[8]
                    plsc.store_scatter(outA, [fvec, chv],
                                       p0 * w[9] + p1 * w[10] + p2 * w[11])
                    plsc.store_scatter(outA, [fvec, chv + C],
                                       p0 * w[12] + p1 * w[13] + p2 * w[14])
                    return 0

                lax.fori_loop(0, C, chan, 0)
            pltpu.sync_copy(outA, ewns_h.at[pl.ds(t * CH_A, CH_A)])
            return 0

        lax.fori_loop(0, TA // NW, a_task, 0)

        def l_task(i, _):
            t = wid + i * NW

            @pl.when(t < TL)
            def _():
                pltpu.sync_copy(idxl_h.at[t], idx_v.at[pl.ds(0, 7)])
                pltpu.sync_copy(wl_h.at[t], w_v.at[pl.ds(0, 7)])
                cps = [pltpu.async_copy(xT_h.at[idx_v.at[j]], rows_v.at[j],
                                        sem) for j in range(7)]
                for cp in cps:
                    cp.wait()

                for f0 in range(0, CH_L, 16):
                    w = [w_v[m, pl.ds(f0, 16)] for m in range(7)]
                    fvec = f0 + lax.iota(jnp.int32, 16)
                    jc = [jnp.full((16,), j, jnp.int32) for j in range(7)]

                    def chan(ch, _):
                        chv = jnp.full((16,), ch, jnp.int32)
                        v = [plsc.load_gather(rows_v, [jc[j], fvec, chv])
                             for j in range(7)]
                        acc = v[0] * w[0]
                        for j in range(1, 7):
                            acc = acc + v[j] * w[j]
                        plsc.store_scatter(outL, [fvec, chv], acc)
                        return 0

                    lax.fori_loop(0, C, chan, 0)
                pltpu.sync_copy(outL, lap_h.at[pl.ds(t * CH_L, CH_L)])

            return 0

        lax.fori_loop(0, (TL + NW - 1) // NW, l_task, 0)

    return body(xT, idxg, wg, idxl, wl)


def _k2(ewns, idxf, wf):
    @functools.partial(
        pl.kernel, mesh=_sc_mesh(),
        out_type=jax.ShapeDtypeStruct((BS * NVP, 2 * C), jnp.float32),
        scratch_types=[
            pltpu.VMEM((6, CH_F), jnp.int32),
            pltpu.VMEM((6, CH_F), jnp.float32),
            pltpu.VMEM((6, CH_F, 2 * C), jnp.float32),
            pltpu.VMEM((CH_F, 2 * C), jnp.float32),
            pltpu.SemaphoreType.DMA,
        ],
        compiler_params=pltpu.CompilerParams(needs_layout_passes=False),
    )
    def body(ewns_h, idxf_h, wf_h, gv_h, idx_v, w_v, rows_v, out_v, sem):
        wid = lax.axis_index("s") * 2 + lax.axis_index("c")

        def task(i, _):
            t = wid + i * NW
            pltpu.sync_copy(idxf_h.at[t], idx_v)
            pltpu.sync_copy(wf_h.at[t], w_v)
            cps = [pltpu.async_copy(ewns_h.at[idx_v.at[j]], rows_v.at[j], sem)
                   for j in range(6)]
            for cp in cps:
                cp.wait()

            for f0 in range(0, CH_F, 16):
                w = [w_v[m, pl.ds(f0, 16)] for m in range(6)]
                fvec = f0 + lax.iota(jnp.int32, 16)
                jc = [jnp.full((16,), j, jnp.int32) for j in range(6)]

                def chan(ch, _):
                    chv = jnp.full((16,), ch, jnp.int32)
                    v = [plsc.load_gather(rows_v, [jc[j], fvec, chv])
                         for j in range(6)]
                    acc = v[0] * w[0]
                    for j in range(1, 6):
                        acc = acc + v[j] * w[j]
                    plsc.store_scatter(out_v, [fvec, chv], acc)
                    return 0

                lax.fori_loop(0, 2 * C, chan, 0)
            pltpu.sync_copy(out_v, gv_h.at[pl.ds(t * CH_F, CH_F)])
            return 0

        lax.fori_loop(0, TF // NW, task, 0)

    return body(ewns, idxf, wf)


def _final_matmul(xT, lap, gv, C0, C1, C23):
    def body(x_ref, l_ref, g_ref, c0_ref, c1_ref, c23_ref, o_ref):
        a = lax.dot_general(c0_ref[...], x_ref[0],
                            (((0,), (1,)), ((), ())),
                            preferred_element_type=jnp.float32)
        b = lax.dot_general(c1_ref[...], l_ref[0],
                            (((0,), (1,)), ((), ())),
                            preferred_element_type=jnp.float32)
        c = lax.dot_general(c23_ref[...], g_ref[0],
                            (((0,), (1,)), ((), ())),
                            preferred_element_type=jnp.float32)
        o_ref[0] = a + b + c

    return pl.pallas_call(
        body,
        grid=(BS, NVP // TV),
        in_specs=[
            pl.BlockSpec((1, TV, 128), lambda b, i: (b, i, 0)),
            pl.BlockSpec((1, TV, 128), lambda b, i: (b, i, 0)),
            pl.BlockSpec((1, TV, 256), lambda b, i: (b, i, 0)),
            pl.BlockSpec((128, 128), lambda b, i: (0, 0)),
            pl.BlockSpec((128, 128), lambda b, i: (0, 0)),
            pl.BlockSpec((256, 128), lambda b, i: (0, 0)),
        ],
        out_specs=pl.BlockSpec((1, 128, TV), lambda b, i: (b, 0, i)),
        out_shape=jax.ShapeDtypeStruct((BS, 128, NVP), jnp.float32),
    )(xT, lap, gv, C0, C1, C23)


def _chunk(arr2d, nrows, chunk):
    k = arr2d.shape[0]
    return arr2d.reshape(k, nrows // chunk, chunk).transpose(1, 0, 2)


def kernel(input, coeffs, G_rows, G_cols, G_vals, L_rows, L_cols, L_vals,
           F_rows, F_cols, F_vals, NS, EW):
    xT = jnp.concatenate(
        [input.transpose(0, 2, 1),
         jnp.ones((BS, NV - NV_PREV, C), jnp.float32)], axis=1)
    xflat = xT.reshape(BS * NV, C)

    gcols = G_cols.reshape(3, NF, 3).transpose(1, 0, 2).reshape(NF, 9).T
    gvals = G_vals.reshape(3, NF, 3).transpose(1, 0, 2).reshape(NF, 9).T
    ga = _chunk(gcols, NF, CH_A)
    idxg = jnp.concatenate([ga, ga + NV], axis=0)
    wg1 = jnp.concatenate(
        [_chunk(gvals, NF, CH_A),
         _chunk(EW.T, NF, CH_A), _chunk(NS.T, NF, CH_A)], axis=1)
    wg = jnp.concatenate([wg1, wg1], axis=0)

    padv = ((0, 0), (0, NVP - NV))
    lc = _chunk(jnp.pad(L_cols.reshape(NV, 7).T, padv), NVP, CH_L)
    idxl = jnp.concatenate([lc, lc + NV], axis=0)
    wl1 = _chunk(jnp.pad(L_vals.reshape(NV, 7).T, padv), NVP, CH_L)
    wl = jnp.concatenate([wl1, wl1], axis=0)

    fc = _chunk(jnp.pad(F_cols.reshape(NV, 6).T, padv), NVP, CH_F)
    idxf = jnp.concatenate([fc, fc + NF], axis=0)
    wf1 = _chunk(jnp.pad(F_vals.reshape(NV, 6).T, padv), NVP, CH_F)
    wf = jnp.concatenate([wf1, wf1], axis=0)

    ewns, lap = _k1(xflat, idxg, wg, idxl, wl)
    gv = _k2(ewns, idxf, wf)

    C4 = coeffs.reshape(C, 4, 128)
    C0 = C4[:, 0, :]
    C1 = C4[:, 1, :]
    C23 = jnp.concatenate([C4[:, 2, :], C4[:, 3, :]], axis=0)

    xpad = jnp.pad(xT, ((0, 0), (0, NVP - NV), (0, 0)))
    out = _final_matmul(xpad, lap.reshape(BS, NVP, C),
                        gv.reshape(BS, NVP, 2 * C), C0, C1, C23)
    return out[:, :, :NV]


# speedup vs baseline: 56.1678x; 3.3222x over previous
import functools
import jax
import jax.numpy as jnp
from jax import lax
from jax.experimental import pallas as pl
from jax.experimental.pallas import tpu as pltpu
from jax.experimental.pallas import tpu_sc as plsc

NV = 40962
NV_PREV = 10242
NF = 81920
C = 128
BS = 2
NVP = 41472
TV = 512

NW = 32
CH_A = 64
CH_L = 64
CH_F = 32
TA = BS * NF // CH_A
NCL = NVP // CH_L
TL = BS * NCL
TF = BS * NVP // CH_F


def _sc_mesh():
    return plsc.VectorSubcoreMesh(core_axis_name="c", subcore_axis_name="s")


def _k1(xT, idxg, wg, idxl, wl):
    @functools.partial(
        pl.kernel, mesh=_sc_mesh(),
        out_type=[
            jax.ShapeDtypeStruct((BS * NF, 2 * C), jnp.float32),
            jax.ShapeDtypeStruct((BS * NVP, C), jnp.float32),
        ],
        scratch_types=[
            pltpu.VMEM((9, CH_A), jnp.int32),
            pltpu.VMEM((CH_A, 16), jnp.float32),
            pltpu.VMEM((9, CH_A, C), jnp.float32),
            pltpu.VMEM((CH_A, 2 * C), jnp.float32),
            pltpu.VMEM((CH_L, C), jnp.float32),
            pltpu.SemaphoreType.DMA,
        ],
        compiler_params=pltpu.CompilerParams(needs_layout_passes=False),
    )
    def body(xT_h, idxg_h, wg_h, idxl_h, wl_h, ewns_h, lap_h,
             idx_v, w_v, rows_v, outA, outL, sem):
        wid = lax.axis_index("s") * 2 + lax.axis_index("c")

        def a_task(i, _):
            t = wid + i * NW
            pltpu.sync_copy(idxg_h.at[t], idx_v)
            pltpu.sync_copy(wg_h.at[t], w_v)
            cps = [pltpu.async_copy(xT_h.at[idx_v.at[j]], rows_v.at[j], sem)
                   for j in range(9)]
            for cp in cps:
                cp.wait()

            def face(f, _):
                wv = w_v[f, :]
                w = [wv[m] for m in range(15)]
                for c0 in range(0, C, 16):
                    v = [rows_v[j, f, pl.ds(c0, 16)] for j in range(9)]
                    p0 = v[0] * w[0] + v[1] * w[1] + v[2] * w[2]
                    p1 = v[3] * w[3] + v[4] * w[4] + v[5] * w[5]
                    p2 = v[6] * w[6] + v[7] * w[7] + v[8] * w[8]
                    outA[f, pl.ds(c0, 16)] = p0 * w[9] + p1 * w[10] + p2 * w[11]
                    outA[f, pl.ds(C + c0, 16)] = (p0 * w[12] + p1 * w[13]
                                                  + p2 * w[14])
                return 0

            lax.fori_loop(0, CH_A, face, 0)
            pltpu.sync_copy(outA, ewns_h.at[pl.ds(t * CH_A, CH_A)])
            return 0

        lax.fori_loop(0, TA // NW, a_task, 0)

        def l_task(i, _):
            t = wid + i * NW

            @pl.when(t < TL)
            def _():
                pltpu.sync_copy(idxl_h.at[t], idx_v.at[pl.ds(0, 7)])
                pltpu.sync_copy(wl_h.at[t], w_v)
                cps = [pltpu.async_copy(xT_h.at[idx_v.at[j]], rows_v.at[j],
                                        sem) for j in range(7)]
                for cp in cps:
                    cp.wait()

                def vert(f, _):
                    wv = w_v[f, :]
                    w = [wv[m] for m in range(7)]
                    for c0 in range(0, C, 16):
                        v = [rows_v[j, f, pl.ds(c0, 16)] for j in range(7)]
                        acc = v[0] * w[0]
                        for j in range(1, 7):
                            acc = acc + v[j] * w[j]
                        outL[f, pl.ds(c0, 16)] = acc
                    return 0

                lax.fori_loop(0, CH_L, vert, 0)
                pltpu.sync_copy(outL, lap_h.at[pl.ds(t * CH_L, CH_L)])

            return 0

        lax.fori_loop(0, (TL + NW - 1) // NW, l_task, 0)

    return body(xT, idxg, wg, idxl, wl)


def _k2(ewns, idxf, wf):
    @functools.partial(
        pl.kernel, mesh=_sc_mesh(),
        out_type=jax.ShapeDtypeStruct((BS * NVP, 2 * C), jnp.float32),
        scratch_types=[
            pltpu.VMEM((6, CH_F), jnp.int32),
            pltpu.VMEM((CH_F, 16), jnp.float32),
            pltpu.VMEM((6, CH_F, 2 * C), jnp.float32),
            pltpu.VMEM((CH_F, 2 * C), jnp.float32),
            pltpu.SemaphoreType.DMA,
        ],
        compiler_params=pltpu.CompilerParams(needs_layout_passes=False),
    )
    def body(ewns_h, idxf_h, wf_h, gv_h, idx_v, w_v, rows_v, out_v, sem):
        wid = lax.axis_index("s") * 2 + lax.axis_index("c")

        def task(i, _):
            t = wid + i * NW
            pltpu.sync_copy(idxf_h.at[t], idx_v)
            pltpu.sync_copy(wf_h.at[t], w_v)
            cps = [pltpu.async_copy(ewns_h.at[idx_v.at[j]], rows_v.at[j], sem)
                   for j in range(6)]
            for cp in cps:
                cp.wait()

            def vert(f, _):
                wv = w_v[f, :]
                w = [wv[m] for m in range(6)]
                for c0 in range(0, 2 * C, 16):
                    v = [rows_v[j, f, pl.ds(c0, 16)] for j in range(6)]
                    acc = v[0] * w[0]
                    for j in range(1, 6):
                        acc = acc + v[j] * w[j]
                    out_v[f, pl.ds(c0, 16)] = acc
                return 0

            lax.fori_loop(0, CH_F, vert, 0)
            pltpu.sync_copy(out_v, gv_h.at[pl.ds(t * CH_F, CH_F)])
            return 0

        lax.fori_loop(0, TF // NW, task, 0)

    return body(ewns, idxf, wf)


def _final_matmul(xT, lap, gv, C0, C1, C23):
    def body(x_ref, l_ref, g_ref, c0_ref, c1_ref, c23_ref, o_ref):
        a = lax.dot_general(c0_ref[...], x_ref[0],
                            (((0,), (1,)), ((), ())),
                            preferred_element_type=jnp.float32)
        b = lax.dot_general(c1_ref[...], l_ref[0],
                            (((0,), (1,)), ((), ())),
                            preferred_element_type=jnp.float32)
        c = lax.dot_general(c23_ref[...], g_ref[0],
                            (((0,), (1,)), ((), ())),
                            preferred_element_type=jnp.float32)
        o_ref[0] = a + b + c

    return pl.pallas_call(
        body,
        grid=(BS, NVP // TV),
        in_specs=[
            pl.BlockSpec((1, TV, 128), lambda b, i: (b, i, 0)),
            pl.BlockSpec((1, TV, 128), lambda b, i: (b, i, 0)),
            pl.BlockSpec((1, TV, 256), lambda b, i: (b, i, 0)),
            pl.BlockSpec((128, 128), lambda b, i: (0, 0)),
            pl.BlockSpec((128, 128), lambda b, i: (0, 0)),
            pl.BlockSpec((256, 128), lambda b, i: (0, 0)),
        ],
        out_specs=pl.BlockSpec((1, 128, TV), lambda b, i: (b, 0, i)),
        out_shape=jax.ShapeDtypeStruct((BS, 128, NVP), jnp.float32),
    )(xT, lap, gv, C0, C1, C23)


def _chunk(arr2d, nrows, chunk):
    k = arr2d.shape[0]
    return arr2d.reshape(k, nrows // chunk, chunk).transpose(1, 0, 2)


def kernel(input, coeffs, G_rows, G_cols, G_vals, L_rows, L_cols, L_vals,
           F_rows, F_cols, F_vals, NS, EW):
    xT = jnp.concatenate(
        [input.transpose(0, 2, 1),
         jnp.ones((BS, NV - NV_PREV, C), jnp.float32)], axis=1)
    xflat = xT.reshape(BS * NV, C)

    gcols = G_cols.reshape(3, NF, 3).transpose(1, 0, 2).reshape(NF, 9)
    gvals = G_vals.reshape(3, NF, 3).transpose(1, 0, 2).reshape(NF, 9)
    ga = _chunk(gcols.T, NF, CH_A)
    idxg = jnp.concatenate([ga, ga + NV], axis=0)
    wg1 = jnp.concatenate(
        [gvals, EW, NS, jnp.zeros((NF, 1), jnp.float32)],
        axis=1).reshape(NF // CH_A, CH_A, 16)
    wg = jnp.concatenate([wg1, wg1], axis=0)

    padv = ((0, 0), (0, NVP - NV))
    lc = _chunk(jnp.pad(L_cols.reshape(NV, 7).T, padv), NVP, CH_L)
    idxl = jnp.concatenate([lc, lc + NV], axis=0)
    wl1 = jnp.pad(L_vals.reshape(NV, 7),
                  ((0, NVP - NV), (0, 9))).reshape(NVP // CH_L, CH_L, 16)
    wl = jnp.concatenate([wl1, wl1], axis=0)

    fc = _chunk(jnp.pad(F_cols.reshape(NV, 6).T, padv), NVP, CH_F)
    idxf = jnp.concatenate([fc, fc + NF], axis=0)
    wf1 = jnp.pad(F_vals.reshape(NV, 6),
                  ((0, NVP - NV), (0, 10))).reshape(NVP // CH_F, CH_F, 16)
    wf = jnp.concatenate([wf1, wf1], axis=0)

    ewns, lap = _k1(xflat, idxg, wg, idxl, wl)
    gv = _k2(ewns, idxf, wf)

    C4 = coeffs.reshape(C, 4, 128)
    C0 = C4[:, 0, :]
    C1 = C4[:, 1, :]
    C23 = jnp.concatenate([C4[:, 2, :], C4[:, 3, :]], axis=0)

    xpad = jnp.pad(xT, ((0, 0), (0, NVP - NV), (0, 0)))
    out = _final_matmul(xpad, lap.reshape(BS, NVP, C),
                        gv.reshape(BS, NVP, 2 * C), C0, C1, C23)
    return out[:, :, :NV]


# speedup vs baseline: 61.6091x; 1.0969x over previous
import functools
import jax
import jax.numpy as jnp
from jax import lax
from jax.experimental import pallas as pl
from jax.experimental.pallas import tpu as pltpu
from jax.experimental.pallas import tpu_sc as plsc

NV = 40962
NV_PREV = 10242
NF = 81920
C = 128
BS = 2
NVP = 41472
TV = 512

NW = 32
CH = 32
TA = BS * NF // CH
TL = BS * NVP // CH
TF = BS * NVP // CH


def _sc_mesh():
    return plsc.VectorSubcoreMesh(core_axis_name="c", subcore_axis_name="s")


def _run_stage(ntasks, wid, meta, issue, drain, compute):
    ntw = -(-ntasks // NW)
    npairs = -(-ntw // 2)

    meta(0, wid)
    issue(0, wid)

    def pair(i2, _):
        te = wid + (2 * i2) * NW
        to = wid + (2 * i2 + 1) * NW
        ten = wid + (2 * i2 + 2) * NW

        @pl.when(to < ntasks)
        def _():
            meta(1, to)
            issue(1, to)

        @pl.when(te < ntasks)
        def _():
            drain(0)
            compute(0, te)

        @pl.when(ten < ntasks)
        def _():
            meta(0, ten)
            issue(0, ten)

        @pl.when(to < ntasks)
        def _():
            drain(1)
            compute(1, to)

        return 0

    lax.fori_loop(0, npairs, pair, 0)


def _k1(xT, idxg, wg, idxl, wl):
    @functools.partial(
        pl.kernel, mesh=_sc_mesh(),
        out_type=[
            jax.ShapeDtypeStruct((BS * NF, 2 * C), jnp.float32),
            jax.ShapeDtypeStruct((BS * NVP, C), jnp.float32),
        ],
        scratch_types=[
            pltpu.VMEM((2, 9, CH), jnp.int32),
            pltpu.VMEM((2, CH, 16), jnp.float32),
            pltpu.VMEM((2, 9, CH, C), jnp.float32),
            pltpu.VMEM((2, CH, 2 * C), jnp.float32),
            pltpu.VMEM((2, CH, C), jnp.float32),
            pltpu.SemaphoreType.DMA,
            pltpu.SemaphoreType.DMA,
        ],
        compiler_params=pltpu.CompilerParams(needs_layout_passes=False),
    )
    def body(xT_h, idxg_h, wg_h, idxl_h, wl_h, ewns_h, lap_h,
             idx2, w2, rows2, outA2, outL2, sg0, sg1):
        wid = lax.axis_index("s") * 2 + lax.axis_index("c")
        sems = [sg0, sg1]

        def metaA(p, t):
            pltpu.sync_copy(idxg_h.at[t], idx2.at[p])
            pltpu.sync_copy(wg_h.at[t], w2.at[p])

        def issueA(p, t):
            for j in range(9):
                pltpu.async_copy(xT_h.at[idx2.at[p, j]], rows2.at[p, j],
                                 sems[p])

        def drainA(p):
            for j in range(9):
                pltpu.make_async_copy(xT_h.at[idx2.at[p, j]],
                                      rows2.at[p, j], sems[p]).wait()

        def computeA(p, t):
            def face(f, _):
                wv = w2[p, f, :]
                w = [wv[m] for m in range(15)]
                for c0 in range(0, C, 16):
                    v = [rows2[p, j, f, pl.ds(c0, 16)] for j in range(9)]
                    p0 = v[0] * w[0] + v[1] * w[1] + v[2] * w[2]
                    p1 = v[3] * w[3] + v[4] * w[4] + v[5] * w[5]
                    p2 = v[6] * w[6] + v[7] * w[7] + v[8] * w[8]
                    outA2[p, f, pl.ds(c0, 16)] = (p0 * w[9] + p1 * w[10]
                                                  + p2 * w[11])
                    outA2[p, f, pl.ds(C + c0, 16)] = (p0 * w[12] + p1 * w[13]
                                                      + p2 * w[14])
                return 0

            lax.fori_loop(0, CH, face, 0)
            pltpu.sync_copy(outA2.at[p], ewns_h.at[pl.ds(t * CH, CH)])

        _run_stage(TA, wid, metaA, issueA, drainA, computeA)

        def metaL(p, t):
            pltpu.sync_copy(idxl_h.at[t], idx2.at[p, pl.ds(0, 7)])
            pltpu.sync_copy(wl_h.at[t], w2.at[p])

        def issueL(p, t):
            for j in range(7):
                pltpu.async_copy(xT_h.at[idx2.at[p, j]], rows2.at[p, j],
                                 sems[p])

        def drainL(p):
            for j in range(7):
                pltpu.make_async_copy(xT_h.at[idx2.at[p, j]],
                                      rows2.at[p, j], sems[p]).wait()

        def computeL(p, t):
            def vert(f, _):
                wv = w2[p, f, :]
                w = [wv[m] for m in range(7)]
                for c0 in range(0, C, 16):
                    v = [rows2[p, j, f, pl.ds(c0, 16)] for j in range(7)]
                    acc = v[0] * w[0]
                    for j in range(1, 7):
                        acc = acc + v[j] * w[j]
                    outL2[p, f, pl.ds(c0, 16)] = acc
                return 0

            lax.fori_loop(0, CH, vert, 0)
            pltpu.sync_copy(outL2.at[p], lap_h.at[pl.ds(t * CH, CH)])

        _run_stage(TL, wid, metaL, issueL, drainL, computeL)

    return body(xT, idxg, wg, idxl, wl)


def _k2(ewns, idxf, wf):
    @functools.partial(
        pl.kernel, mesh=_sc_mesh(),
        out_type=jax.ShapeDtypeStruct((BS * NVP, 2 * C), jnp.float32),
        scratch_types=[
            pltpu.VMEM((2, 6, CH), jnp.int32),
            pltpu.VMEM((2, CH, 16), jnp.float32),
            pltpu.VMEM((2, 6, CH, 2 * C), jnp.float32),
            pltpu.VMEM((2, CH, 2 * C), jnp.float32),
            pltpu.SemaphoreType.DMA,
            pltpu.SemaphoreType.DMA,
        ],
        compiler_params=pltpu.CompilerParams(needs_layout_passes=False),
    )
    def body(ewns_h, idxf_h, wf_h, gv_h, idx2, w2, rows2, out2, sg0, sg1):
        wid = lax.axis_index("s") * 2 + lax.axis_index("c")
        sems = [sg0, sg1]

        def metaF(p, t):
            pltpu.sync_copy(idxf_h.at[t], idx2.at[p])
            pltpu.sync_copy(wf_h.at[t], w2.at[p])

        def issueF(p, t):
            for j in range(6):
                pltpu.async_copy(ewns_h.at[idx2.at[p, j]], rows2.at[p, j],
                                 sems[p])

        def drainF(p):
            for j in range(6):
                pltpu.make_async_copy(ewns_h.at[idx2.at[p, j]],
                                      rows2.at[p, j], sems[p]).wait()

        def computeF(p, t):
            def vert(f, _):
                wv = w2[p, f, :]
                w = [wv[m] for m in range(6)]
                for c0 in range(0, 2 * C, 16):
                    v = [rows2[p, j, f, pl.ds(c0, 16)] for j in range(6)]
                    acc = v[0] * w[0]
                    for j in range(1, 6):
                        acc = acc + v[j] * w[j]
                    out2[p, f, pl.ds(c0, 16)] = acc
                return 0

            lax.fori_loop(0, CH, vert, 0)
            pltpu.sync_copy(out2.at[p], gv_h.at[pl.ds(t * CH, CH)])

        _run_stage(TF, wid, metaF, issueF, drainF, computeF)

    return body(ewns, idxf, wf)


def _final_matmul(xT, lap, gv, C0, C1, C23):
    def body(x_ref, l_ref, g_ref, c0_ref, c1_ref, c23_ref, o_ref):
        a = lax.dot_general(c0_ref[...], x_ref[0],
                            (((0,), (1,)), ((), ())),
                            preferred_element_type=jnp.float32)
        b = lax.dot_general(c1_ref[...], l_ref[0],
                            (((0,), (1,)), ((), ())),
                            preferred_element_type=jnp.float32)
        c = lax.dot_general(c23_ref[...], g_ref[0],
                            (((0,), (1,)), ((), ())),
                            preferred_element_type=jnp.float32)
        o_ref[0] = a + b + c

    return pl.pallas_call(
        body,
        grid=(BS, NVP // TV),
        in_specs=[
            pl.BlockSpec((1, TV, 128), lambda b, i: (b, i, 0)),
            pl.BlockSpec((1, TV, 128), lambda b, i: (b, i, 0)),
            pl.BlockSpec((1, TV, 256), lambda b, i: (b, i, 0)),
            pl.BlockSpec((128, 128), lambda b, i: (0, 0)),
            pl.BlockSpec((128, 128), lambda b, i: (0, 0)),
            pl.BlockSpec((256, 128), lambda b, i: (0, 0)),
        ],
        out_specs=pl.BlockSpec((1, 128, TV), lambda b, i: (b, 0, i)),
        out_shape=jax.ShapeDtypeStruct((BS, 128, NVP), jnp.float32),
    )(xT, lap, gv, C0, C1, C23)


def _chunk(arr2d, nrows, chunk):
    k = arr2d.shape[0]
    return arr2d.reshape(k, nrows // chunk, chunk).transpose(1, 0, 2)


def kernel(input, coeffs, G_rows, G_cols, G_vals, L_rows, L_cols, L_vals,
           F_rows, F_cols, F_vals, NS, EW):
    xT = jnp.concatenate(
        [input.transpose(0, 2, 1),
         jnp.ones((BS, NV - NV_PREV, C), jnp.float32)], axis=1)
    xflat = xT.reshape(BS * NV, C)

    gcols = G_cols.reshape(3, NF, 3).transpose(1, 0, 2).reshape(NF, 9)
    gvals = G_vals.reshape(3, NF, 3).transpose(1, 0, 2).reshape(NF, 9)
    ga = _chunk(gcols.T, NF, CH)
    idxg = jnp.concatenate([ga, ga + NV], axis=0)
    wg1 = jnp.concatenate(
        [gvals, EW, NS, jnp.zeros((NF, 1), jnp.float32)],
        axis=1).reshape(NF // CH, CH, 16)
    wg = jnp.concatenate([wg1, wg1], axis=0)

    padv = ((0, 0), (0, NVP - NV))
    lc = _chunk(jnp.pad(L_cols.reshape(NV, 7).T, padv), NVP, CH)
    idxl = jnp.concatenate([lc, lc + NV], axis=0)
    wl1 = jnp.pad(L_vals.reshape(NV, 7),
                  ((0, NVP - NV), (0, 9))).reshape(NVP // CH, CH, 16)
    wl = jnp.concatenate([wl1, wl1], axis=0)

    fc = _chunk(jnp.pad(F_cols.reshape(NV, 6).T, padv), NVP, CH)
    idxf = jnp.concatenate([fc, fc + NF], axis=0)
    wf1 = jnp.pad(F_vals.reshape(NV, 6),
                  ((0, NVP - NV), (0, 10))).reshape(NVP // CH, CH, 16)
    wf = jnp.concatenate([wf1, wf1], axis=0)

    ewns, lap = _k1(xflat, idxg, wg, idxl, wl)
    gv = _k2(ewns, idxf, wf)

    C4 = coeffs.reshape(C, 4, 128)
    C0 = C4[:, 0, :]
    C1 = C4[:, 1, :]
    C23 = jnp.concatenate([C4[:, 2, :], C4[:, 3, :]], axis=0)

    xpad = jnp.pad(xT, ((0, 0), (0, NVP - NV), (0, 0)))
    out = _final_matmul(xpad, lap.reshape(BS, NVP, C),
                        gv.reshape(BS, NVP, 2 * C), C0, C1, C23)
    return out[:, :, :NV]


# speedup vs baseline: 72.0578x; 1.1696x over previous
import functools
import jax
import jax.numpy as jnp
from jax import lax
from jax.experimental import pallas as pl
from jax.experimental.pallas import tpu as pltpu
from jax.experimental.pallas import tpu_sc as plsc

NV = 40962
NV_PREV = 10242
NF = 81920
C = 128
BS = 2
NVP = 41472
TV = 512

NW = 32
CH = 32
TA = BS * NF // CH
TL = BS * NVP // CH
TF = BS * NVP // CH


def _sc_mesh():
    return plsc.VectorSubcoreMesh(core_axis_name="c", subcore_axis_name="s")


def _run_stage(ntasks, wid, meta_idx, meta_w, wait_idx, wait_w, issue,
               drain, compute, drain_out):
    ntw = -(-ntasks // NW)
    npairs = -(-ntw // 2)

    meta_idx(0, wid)
    meta_w(0, wid)
    wait_idx(0)
    issue(0, wid)

    @pl.when(wid + NW < ntasks)
    def _():
        meta_idx(1, wid + NW)
        meta_w(1, wid + NW)

    def pair(i2, _):
        te = wid + (2 * i2) * NW
        to = te + NW
        te2 = to + NW
        to2 = te2 + NW

        @pl.when(te < ntasks)
        def _():
            drain(0)

            @pl.when(to < ntasks)
            def _():
                wait_idx(1)
                issue(1, to)

            @pl.when(te2 < ntasks)
            def _():
                meta_idx(0, te2)

            @pl.when(te >= wid + 2 * NW)
            def _():
                drain_out(0)

            wait_w(0)
            compute(0, te)

            @pl.when(te2 < ntasks)
            def _():
                meta_w(0, te2)

        @pl.when(to < ntasks)
        def _():
            drain(1)

            @pl.when(te2 < ntasks)
            def _():
                wait_idx(0)
                issue(0, te2)

            @pl.when(to2 < ntasks)
            def _():
                meta_idx(1, to2)

            @pl.when(to >= wid + 2 * NW)
            def _():
                drain_out(1)

            wait_w(1)
            compute(1, to)

            @pl.when(to2 < ntasks)
            def _():
                meta_w(1, to2)

        return 0

    lax.fori_loop(0, npairs, pair, 0)

    drain_out(0)
    @pl.when(wid + NW < ntasks)
    def _():
        drain_out(1)


def _k1(xT, idxg, wg, idxl, wl):
    @functools.partial(
        pl.kernel, mesh=_sc_mesh(),
        out_type=[
            jax.ShapeDtypeStruct((BS * NF, 2 * C), jnp.float32),
            jax.ShapeDtypeStruct((BS * NVP, C), jnp.float32),
        ],
        scratch_types=[
            pltpu.VMEM((2, 9, CH), jnp.int32),
            pltpu.VMEM((2, CH, 16), jnp.float32),
            pltpu.VMEM((2, 9, CH, C), jnp.float32),
            pltpu.VMEM((2, CH, 2 * C), jnp.float32),
            pltpu.VMEM((2, CH, C), jnp.float32),
            pltpu.SemaphoreType.DMA,
            pltpu.SemaphoreType.DMA,
            pltpu.SemaphoreType.DMA,
            pltpu.SemaphoreType.DMA,
            pltpu.SemaphoreType.DMA,
            pltpu.SemaphoreType.DMA,
            pltpu.SemaphoreType.DMA,
            pltpu.SemaphoreType.DMA,
        ],
        compiler_params=pltpu.CompilerParams(needs_layout_passes=False),
    )
    def body(xT_h, idxg_h, wg_h, idxl_h, wl_h, ewns_h, lap_h,
             idx2, w2, rows2, outA2, outL2,
             sg0, sg1, si0, si1, sw0, sw1, so0, so1):
        wid = lax.axis_index("s") * 2 + lax.axis_index("c")
        sg = [sg0, sg1]
        si = [si0, si1]
        sw = [sw0, sw1]
        so = [so0, so1]

        def metaA_idx(p, t):
            pltpu.async_copy(idxg_h.at[t], idx2.at[p], si[p])

        def metaA_w(p, t):
            pltpu.async_copy(wg_h.at[t], w2.at[p], sw[p])

        def wait_idxA(p):
            pltpu.make_async_copy(idxg_h.at[0], idx2.at[p], si[p]).wait()

        def wait_wA(p):
            pltpu.make_async_copy(wg_h.at[0], w2.at[p], sw[p]).wait()

        def issueA(p, t):
            for j in range(9):
                pltpu.async_copy(xT_h.at[idx2.at[p, j]], rows2.at[p, j],
                                 sg[p])

        def drainA(p):
            for j in range(9):
                pltpu.make_async_copy(xT_h.at[idx2.at[p, j]],
                                      rows2.at[p, j], sg[p]).wait()

        def computeA(p, t):
            def face(f, _):
                wv = w2[p, f, :]
                w = [wv[m] for m in range(15)]
                for c0 in range(0, C, 16):
                    v = [rows2[p, j, f, pl.ds(c0, 16)] for j in range(9)]
                    p0 = v[0] * w[0] + v[1] * w[1] + v[2] * w[2]
                    p1 = v[3] * w[3] + v[4] * w[4] + v[5] * w[5]
                    p2 = v[6] * w[6] + v[7] * w[7] + v[8] * w[8]
                    outA2[p, f, pl.ds(c0, 16)] = (p0 * w[9] + p1 * w[10]
                                                  + p2 * w[11])
                    outA2[p, f, pl.ds(C + c0, 16)] = (p0 * w[12] + p1 * w[13]
                                                      + p2 * w[14])
                return 0

            lax.fori_loop(0, CH, face, 0, unroll=2)
            pltpu.async_copy(outA2.at[p], ewns_h.at[pl.ds(t * CH, CH)], so[p])

        def drain_outA(p):
            pltpu.make_async_copy(outA2.at[p], ewns_h.at[pl.ds(0, CH)],
                                  so[p]).wait()

        _run_stage(TA, wid, metaA_idx, metaA_w, wait_idxA, wait_wA,
                   issueA, drainA, computeA, drain_outA)

        def metaL_idx(p, t):
            pltpu.async_copy(idxl_h.at[t], idx2.at[p, pl.ds(0, 7)], si[p])

        def metaL_w(p, t):
            pltpu.async_copy(wl_h.at[t], w2.at[p], sw[p])

        def wait_idxL(p):
            pltpu.make_async_copy(idxl_h.at[0], idx2.at[p, pl.ds(0, 7)],
                                  si[p]).wait()

        def wait_wL(p):
            pltpu.make_async_copy(wl_h.at[0], w2.at[p], sw[p]).wait()

        def issueL(p, t):
            for j in range(7):
                pltpu.async_copy(xT_h.at[idx2.at[p, j]], rows2.at[p, j],
                                 sg[p])

        def drainL(p):
            for j in range(7):
                pltpu.make_async_copy(xT_h.at[idx2.at[p, j]],
                                      rows2.at[p, j], sg[p]).wait()

        def computeL(p, t):
            def vert(f, _):
                wv = w2[p, f, :]
                w = [wv[m] for m in range(7)]
                for c0 in range(0, C, 16):
                    v = [rows2[p, j, f, pl.ds(c0, 16)] for j in range(7)]
                    acc = v[0] * w[0]
                    for j in range(1, 7):
                        acc = acc + v[j] * w[j]
                    outL2[p, f, pl.ds(c0, 16)] = acc
                return 0

            lax.fori_loop(0, CH, vert, 0, unroll=2)
            pltpu.async_copy(outL2.at[p], lap_h.at[pl.ds(t * CH, CH)], so[p])

        def drain_outL(p):
            pltpu.make_async_copy(outL2.at[p], lap_h.at[pl.ds(0, CH)],
                                  so[p]).wait()

        _run_stage(TL, wid, metaL_idx, metaL_w, wait_idxL, wait_wL,
                   issueL, drainL, computeL, drain_outL)

    return body(xT, idxg, wg, idxl, wl)


def _k2(ewns, idxf, wf):
    @functools.partial(
        pl.kernel, mesh=_sc_mesh(),
        out_type=jax.ShapeDtypeStruct((BS * NVP, 2 * C), jnp.float32),
        scratch_types=[
            pltpu.VMEM((2, 6, CH), jnp.int32),
            pltpu.VMEM((2, CH, 16), jnp.float32),
            pltpu.VMEM((2, 6, CH, 2 * C), jnp.float32),
            pltpu.VMEM((2, CH, 2 * C), jnp.float32),
            pltpu.SemaphoreType.DMA,
            pltpu.SemaphoreType.DMA,
            pltpu.SemaphoreType.DMA,
            pltpu.SemaphoreType.DMA,
            pltpu.SemaphoreType.DMA,
            pltpu.SemaphoreType.DMA,
            pltpu.SemaphoreType.DMA,
            pltpu.SemaphoreType.DMA,
        ],
        compiler_params=pltpu.CompilerParams(needs_layout_passes=False),
    )
    def body(ewns_h, idxf_h, wf_h, gv_h, idx2, w2, rows2, out2,
             sg0, sg1, si0, si1, sw0, sw1, so0, so1):
        wid = lax.axis_index("s") * 2 + lax.axis_index("c")
        sg = [sg0, sg1]
        si = [si0, si1]
        sw = [sw0, sw1]
        so = [so0, so1]

        def metaF_idx(p, t):
            pltpu.async_copy(idxf_h.at[t], idx2.at[p], si[p])

        def metaF_w(p, t):
            pltpu.async_copy(wf_h.at[t], w2.at[p], sw[p])

        def wait_idxF(p):
            pltpu.make_async_copy(idxf_h.at[0], idx2.at[p], si[p]).wait()

        def wait_wF(p):
            pltpu.make_async_copy(wf_h.at[0], w2.at[p], sw[p]).wait()

        def issueF(p, t):
            for j in range(6):
                pltpu.async_copy(ewns_h.at[idx2.at[p, j]], rows2.at[p, j],
                                 sg[p])

        def drainF(p):
            for j in range(6):
                pltpu.make_async_copy(ewns_h.at[idx2.at[p, j]],
                                      rows2.at[p, j], sg[p]).wait()

        def computeF(p, t):
            def vert(f, _):
                wv = w2[p, f, :]
                w = [wv[m] for m in range(6)]
                for c0 in range(0, 2 * C, 16):
                    v = [rows2[p, j, f, pl.ds(c0, 16)] for j in range(6)]
                    acc = v[0] * w[0]
                    for j in range(1, 6):
                        acc = acc + v[j] * w[j]
                    out2[p, f, pl.ds(c0, 16)] = acc
                return 0

            lax.fori_loop(0, CH, vert, 0, unroll=2)
            pltpu.async_copy(out2.at[p], gv_h.at[pl.ds(t * CH, CH)], so[p])

        def drain_outF(p):
            pltpu.make_async_copy(out2.at[p], gv_h.at[pl.ds(0, CH)],
                                  so[p]).wait()

        _run_stage(TF, wid, metaF_idx, metaF_w, wait_idxF, wait_wF,
                   issueF, drainF, computeF, drain_outF)

    return body(ewns, idxf, wf)


def _final_matmul(xT, lap, gv, C0, C1, C23):
    def body(x_ref, l_ref, g_ref, c0_ref, c1_ref, c23_ref, o_ref):
        a = lax.dot_general(c0_ref[...], x_ref[0],
                            (((0,), (1,)), ((), ())),
                            preferred_element_type=jnp.float32)
        b = lax.dot_general(c1_ref[...], l_ref[0],
                            (((0,), (1,)), ((), ())),
                            preferred_element_type=jnp.float32)
        c = lax.dot_general(c23_ref[...], g_ref[0],
                            (((0,), (1,)), ((), ())),
                            preferred_element_type=jnp.float32)
        o_ref[0] = a + b + c

    return pl.pallas_call(
        body,
        grid=(BS, NVP // TV),
        in_specs=[
            pl.BlockSpec((1, TV, 128), lambda b, i: (b, i, 0)),
            pl.BlockSpec((1, TV, 128), lambda b, i: (b, i, 0)),
            pl.BlockSpec((1, TV, 256), lambda b, i: (b, i, 0)),
            pl.BlockSpec((128, 128), lambda b, i: (0, 0)),
            pl.BlockSpec((128, 128), lambda b, i: (0, 0)),
            pl.BlockSpec((256, 128), lambda b, i: (0, 0)),
        ],
        out_specs=pl.BlockSpec((1, 128, TV), lambda b, i: (b, 0, i)),
        out_shape=jax.ShapeDtypeStruct((BS, 128, NV), jnp.float32),
    )(xT, lap, gv, C0, C1, C23)


def _chunk(arr2d, nrows, chunk):
    k = arr2d.shape[0]
    return arr2d.reshape(k, nrows // chunk, chunk).transpose(1, 0, 2)


def kernel(input, coeffs, G_rows, G_cols, G_vals, L_rows, L_cols, L_vals,
           F_rows, F_cols, F_vals, NS, EW):
    xT = jnp.concatenate(
        [input.transpose(0, 2, 1),
         jnp.ones((BS, NVP - NV_PREV, C), jnp.float32)], axis=1)
    xflat = xT.reshape(BS * NVP, C)

    gcols = G_cols.reshape(3, NF, 3).transpose(1, 0, 2).reshape(NF, 9)
    gvals = G_vals.reshape(3, NF, 3).transpose(1, 0, 2).reshape(NF, 9)
    ga = _chunk(gcols.T, NF, CH)
    idxg = jnp.concatenate([ga, ga + NVP], axis=0)
    wg1 = jnp.concatenate(
        [gvals, EW, NS, jnp.zeros((NF, 1), jnp.float32)],
        axis=1).reshape(NF // CH, CH, 16)
    wg = jnp.concatenate([wg1, wg1], axis=0)

    padv = ((0, 0), (0, NVP - NV))
    lc = _chunk(jnp.pad(L_cols.reshape(NV, 7).T, padv), NVP, CH)
    idxl = jnp.concatenate([lc, lc + NVP], axis=0)
    wl1 = jnp.pad(L_vals.reshape(NV, 7),
                  ((0, NVP - NV), (0, 9))).reshape(NVP // CH, CH, 16)
    wl = jnp.concatenate([wl1, wl1], axis=0)

    fc = _chunk(jnp.pad(F_cols.reshape(NV, 6).T, padv), NVP, CH)
    idxf = jnp.concatenate([fc, fc + NF], axis=0)
    wf1 = jnp.pad(F_vals.reshape(NV, 6),
                  ((0, NVP - NV), (0, 10))).reshape(NVP // CH, CH, 16)
    wf = jnp.concatenate([wf1, wf1], axis=0)

    ewns, lap = _k1(xflat, idxg, wg, idxl, wl)
    gv = _k2(ewns, idxf, wf)

    C4 = coeffs.reshape(C, 4, 128)
    C0 = C4[:, 0, :]
    C1 = C4[:, 1, :]
    C23 = jnp.concatenate([C4[:, 2, :], C4[:, 3, :]], axis=0)

    return _final_matmul(xT, lap.reshape(BS, NVP, C),
                         gv.reshape(BS, NVP, 2 * C), C0, C1, C23)


# speedup vs baseline: 78.8819x; 1.0947x over previous
import functools
import jax
import jax.numpy as jnp
from jax import lax
from jax.experimental import pallas as pl
from jax.experimental.pallas import tpu as pltpu
from jax.experimental.pallas import tpu_sc as plsc

NV = 40962
NV_PREV = 10242
NF = 81920
C = 128
BS = 2
NVP = 41472
TV = 512

NW = 32
CH = 32
TA = BS * NF // CH
TL = BS * NVP // CH
TF = BS * NVP // CH


def _sc_mesh():
    return plsc.VectorSubcoreMesh(core_axis_name="c", subcore_axis_name="s")


def _run_stage(ntasks, wid, meta_idx, meta_w, wait_idx, wait_w, issue,
               drain, compute, drain_out):
    ntw = -(-ntasks // NW)
    npairs = -(-ntw // 2)

    meta_idx(0, wid)
    meta_w(0, wid)
    wait_idx(0)
    issue(0, wid)

    @pl.when(wid + NW < ntasks)
    def _():
        meta_idx(1, wid + NW)
        meta_w(1, wid + NW)

    def pair(i2, _):
        te = wid + (2 * i2) * NW
        to = te + NW
        te2 = to + NW
        to2 = te2 + NW

        @pl.when(te < ntasks)
        def _():
            drain(0)

            @pl.when(to < ntasks)
            def _():
                wait_idx(1)
                issue(1, to)

            @pl.when(te2 < ntasks)
            def _():
                meta_idx(0, te2)

            @pl.when(te >= wid + 2 * NW)
            def _():
                drain_out(0)

            wait_w(0)
            compute(0, te)

            @pl.when(te2 < ntasks)
            def _():
                meta_w(0, te2)

        @pl.when(to < ntasks)
        def _():
            drain(1)

            @pl.when(te2 < ntasks)
            def _():
                wait_idx(0)
                issue(0, te2)

            @pl.when(to2 < ntasks)
            def _():
                meta_idx(1, to2)

            @pl.when(to >= wid + 2 * NW)
            def _():
                drain_out(1)

            wait_w(1)
            compute(1, to)

            @pl.when(to2 < ntasks)
            def _():
                meta_w(1, to2)

        return 0

    lax.fori_loop(0, npairs, pair, 0)

    drain_out(0)
    @pl.when(wid + NW < ntasks)
    def _():
        drain_out(1)


def _k1(xT, idxg, wg, idxl, wl):
    @functools.partial(
        pl.kernel, mesh=_sc_mesh(),
        out_type=[
            jax.ShapeDtypeStruct((BS * NF, 2 * C), jnp.float32),
            jax.ShapeDtypeStruct((BS * NVP, C), jnp.float32),
        ],
        scratch_types=[
            pltpu.VMEM((2, 9, CH), jnp.int32),
            pltpu.VMEM((2, CH, C), jnp.float32),
            pltpu.VMEM((2, 9, CH, C), jnp.float32),
            pltpu.VMEM((2, CH, 2 * C), jnp.float32),
            pltpu.VMEM((2, CH, C), jnp.float32),
            pltpu.SemaphoreType.DMA,
            pltpu.SemaphoreType.DMA,
            pltpu.SemaphoreType.DMA,
            pltpu.SemaphoreType.DMA,
            pltpu.SemaphoreType.DMA,
            pltpu.SemaphoreType.DMA,
            pltpu.SemaphoreType.DMA,
            pltpu.SemaphoreType.DMA,
        ],
        compiler_params=pltpu.CompilerParams(needs_layout_passes=False),
    )
    def body(xT_h, idxg_h, wg_h, idxl_h, wl_h, ewns_h, lap_h,
             idx2, w2, rows2, outA2, outL2,
             sg0, sg1, si0, si1, sw0, sw1, so0, so1):
        wid = lax.axis_index("s") * 2 + lax.axis_index("c")
        sg = [sg0, sg1]
        si = [si0, si1]
        sw = [sw0, sw1]
        so = [so0, so1]

        def metaA_idx(p, t):
            for j in range(9):
                pltpu.async_copy(idxg_h.at[pl.ds(j * 2 * NF + t * CH, CH)],
                                 idx2.at[p, j], si[p])

        def metaA_w(p, t):
            tw = lax.rem(t, NF // CH)
            pltpu.async_copy(wg_h.at[pl.ds(tw * CH, CH)], w2.at[p], sw[p])

        def wait_idxA(p):
            for j in range(9):
                pltpu.make_async_copy(idxg_h.at[pl.ds(0, CH)],
                                      idx2.at[p, j], si[p]).wait()

        def wait_wA(p):
            pltpu.make_async_copy(wg_h.at[pl.ds(0, CH)], w2.at[p],
                                  sw[p]).wait()

        def issueA(p, t):
            for j in range(9):
                pltpu.async_copy(xT_h.at[idx2.at[p, j]], rows2.at[p, j],
                                 sg[p])

        def drainA(p):
            for j in range(9):
                pltpu.make_async_copy(xT_h.at[idx2.at[p, j]],
                                      rows2.at[p, j], sg[p]).wait()

        def computeA(p, t):
            def face(f, _):
                wv = w2[p, f, pl.ds(0, 16)]
                w = [wv[m] for m in range(15)]
                for c0 in range(0, C, 16):
                    v = [rows2[p, j, f, pl.ds(c0, 16)] for j in range(9)]
                    p0 = v[0] * w[0] + v[1] * w[1] + v[2] * w[2]
                    p1 = v[3] * w[3] + v[4] * w[4] + v[5] * w[5]
                    p2 = v[6] * w[6] + v[7] * w[7] + v[8] * w[8]
                    outA2[p, f, pl.ds(c0, 16)] = (p0 * w[9] + p1 * w[10]
                                                  + p2 * w[11])
                    outA2[p, f, pl.ds(C + c0, 16)] = (p0 * w[12] + p1 * w[13]
                                                      + p2 * w[14])
                return 0

            lax.fori_loop(0, CH, face, 0, unroll=2)
            pltpu.async_copy(outA2.at[p], ewns_h.at[pl.ds(t * CH, CH)], so[p])

        def drain_outA(p):
            pltpu.make_async_copy(outA2.at[p], ewns_h.at[pl.ds(0, CH)],
                                  so[p]).wait()

        _run_stage(TA, wid, metaA_idx, metaA_w, wait_idxA, wait_wA,
                   issueA, drainA, computeA, drain_outA)

        def metaL_idx(p, t):
            for j in range(7):
                pltpu.async_copy(idxl_h.at[pl.ds(j * 2 * NVP + t * CH, CH)],
                                 idx2.at[p, j], si[p])

        def metaL_w(p, t):
            tw = lax.rem(t, NVP // CH)
            pltpu.async_copy(wl_h.at[pl.ds(tw * CH, CH)], w2.at[p], sw[p])

        def wait_idxL(p):
            for j in range(7):
                pltpu.make_async_copy(idxl_h.at[pl.ds(0, CH)],
                                      idx2.at[p, j], si[p]).wait()

        def wait_wL(p):
            pltpu.make_async_copy(wl_h.at[pl.ds(0, CH)], w2.at[p],
                                  sw[p]).wait()

        def issueL(p, t):
            for j in range(7):
                pltpu.async_copy(xT_h.at[idx2.at[p, j]], rows2.at[p, j],
                                 sg[p])

        def drainL(p):
            for j in range(7):
                pltpu.make_async_copy(xT_h.at[idx2.at[p, j]],
                                      rows2.at[p, j], sg[p]).wait()

        def computeL(p, t):
            def vert(f, _):
                wv = w2[p, f, pl.ds(0, 16)]
                w = [wv[m] for m in range(7)]
                for c0 in range(0, C, 16):
                    v = [rows2[p, j, f, pl.ds(c0, 16)] for j in range(7)]
                    acc = v[0] * w[0]
                    for j in range(1, 7):
                        acc = acc + v[j] * w[j]
                    outL2[p, f, pl.ds(c0, 16)] = acc
                return 0

            lax.fori_loop(0, CH, vert, 0, unroll=2)
            pltpu.async_copy(outL2.at[p], lap_h.at[pl.ds(t * CH, CH)], so[p])

        def drain_outL(p):
            pltpu.make_async_copy(outL2.at[p], lap_h.at[pl.ds(0, CH)],
                                  so[p]).wait()

        _run_stage(TL, wid, metaL_idx, metaL_w, wait_idxL, wait_wL,
                   issueL, drainL, computeL, drain_outL)

    return body(xT, idxg, wg, idxl, wl)


def _k2(ewns, idxf, wf):
    @functools.partial(
        pl.kernel, mesh=_sc_mesh(),
        out_type=jax.ShapeDtypeStruct((BS * NVP, 2 * C), jnp.float32),
        scratch_types=[
            pltpu.VMEM((2, 6, CH), jnp.int32),
            pltpu.VMEM((2, CH, C), jnp.float32),
            pltpu.VMEM((2, 6, CH, 2 * C), jnp.float32),
            pltpu.VMEM((2, CH, 2 * C), jnp.float32),
            pltpu.SemaphoreType.DMA,
            pltpu.SemaphoreType.DMA,
            pltpu.SemaphoreType.DMA,
            pltpu.SemaphoreType.DMA,
            pltpu.SemaphoreType.DMA,
            pltpu.SemaphoreType.DMA,
            pltpu.SemaphoreType.DMA,
            pltpu.SemaphoreType.DMA,
        ],
        compiler_params=pltpu.CompilerParams(needs_layout_passes=False),
    )
    def body(ewns_h, idxf_h, wf_h, gv_h, idx2, w2, rows2, out2,
             sg0, sg1, si0, si1, sw0, sw1, so0, so1):
        wid = lax.axis_index("s") * 2 + lax.axis_index("c")
        sg = [sg0, sg1]
        si = [si0, si1]
        sw = [sw0, sw1]
        so = [so0, so1]

        def metaF_idx(p, t):
            for j in range(6):
                pltpu.async_copy(idxf_h.at[pl.ds(j * 2 * NVP + t * CH, CH)],
                                 idx2.at[p, j], si[p])

        def metaF_w(p, t):
            tw = lax.rem(t, NVP // CH)
            pltpu.async_copy(wf_h.at[pl.ds(tw * CH, CH)], w2.at[p], sw[p])

        def wait_idxF(p):
            for j in range(6):
                pltpu.make_async_copy(idxf_h.at[pl.ds(0, CH)],
                                      idx2.at[p, j], si[p]).wait()

        def wait_wF(p):
            pltpu.make_async_copy(wf_h.at[pl.ds(0, CH)], w2.at[p],
                                  sw[p]).wait()

        def issueF(p, t):
            for j in range(6):
                pltpu.async_copy(ewns_h.at[idx2.at[p, j]], rows2.at[p, j],
                                 sg[p])

        def drainF(p):
            for j in range(6):
                pltpu.make_async_copy(ewns_h.at[idx2.at[p, j]],
                                      rows2.at[p, j], sg[p]).wait()

        def computeF(p, t):
            def vert(f, _):
                wv = w2[p, f, pl.ds(0, 16)]
                w = [wv[m] for m in range(6)]
                for c0 in range(0, 2 * C, 16):
                    v = [rows2[p, j, f, pl.ds(c0, 16)] for j in range(6)]
                    acc = v[0] * w[0]
                    for j in range(1, 6):
                        acc = acc + v[j] * w[j]
                    out2[p, f, pl.ds(c0, 16)] = acc
                return 0

            lax.fori_loop(0, CH, vert, 0, unroll=2)
            pltpu.async_copy(out2.at[p], gv_h.at[pl.ds(t * CH, CH)], so[p])

        def drain_outF(p):
            pltpu.make_async_copy(out2.at[p], gv_h.at[pl.ds(0, CH)],
                                  so[p]).wait()

        _run_stage(TF, wid, metaF_idx, metaF_w, wait_idxF, wait_wF,
                   issueF, drainF, computeF, drain_outF)

    return body(ewns, idxf, wf)


def _final_matmul(xT, lap, gv, C0, C1, C23):
    def body(x_ref, l_ref, g_ref, c0_ref, c1_ref, c23_ref, o_ref):
        a = lax.dot_general(c0_ref[...], x_ref[0],
                            (((0,), (1,)), ((), ())),
                            preferred_element_type=jnp.float32)
        b = lax.dot_general(c1_ref[...], l_ref[0],
                            (((0,), (1,)), ((), ())),
                            preferred_element_type=jnp.float32)
        c = lax.dot_general(c23_ref[...], g_ref[0],
                            (((0,), (1,)), ((), ())),
                            preferred_element_type=jnp.float32)
        o_ref[0] = a + b + c

    return pl.pallas_call(
        body,
        grid=(BS, NVP // TV),
        in_specs=[
            pl.BlockSpec((1, TV, 128), lambda b, i: (b, i, 0)),
            pl.BlockSpec((1, TV, 128), lambda b, i: (b, i, 0)),
            pl.BlockSpec((1, TV, 256), lambda b, i: (b, i, 0)),
            pl.BlockSpec((128, 128), lambda b, i: (0, 0)),
            pl.BlockSpec((128, 128), lambda b, i: (0, 0)),
            pl.BlockSpec((256, 128), lambda b, i: (0, 0)),
        ],
        out_specs=pl.BlockSpec((1, 128, TV), lambda b, i: (b, 0, i)),
        out_shape=jax.ShapeDtypeStruct((BS, 128, NV), jnp.float32),
    )(xT, lap, gv, C0, C1, C23)


def _strided_rows(flat, k, n):
    rows = [lax.slice(flat, (j,), (j + (n - 1) * k + 1,), (k,))
            for j in range(k)]
    return jnp.stack(rows, axis=0)


def kernel(input, coeffs, G_rows, G_cols, G_vals, L_rows, L_cols, L_vals,
           F_rows, F_cols, F_vals, NS, EW):
    xT = jnp.concatenate(
        [input.transpose(0, 2, 1),
         jnp.ones((BS, NVP - NV_PREV, C), jnp.float32)], axis=1)
    xflat = xT.reshape(BS * NVP, C)

    padv = ((0, 0), (0, NVP - NV))

    gt = jnp.concatenate(
        [_strided_rows(G_cols.reshape(3, 3 * NF)[d], 3, NF)
         for d in range(3)], axis=0)
    idxg = jnp.concatenate([gt, gt + NVP], axis=1).reshape(-1)
    gw = jnp.concatenate(
        [_strided_rows(G_vals.reshape(3, 3 * NF)[d], 3, NF)
         for d in range(3)], axis=0)
    wg = jnp.concatenate(
        [gw.T, EW, NS, jnp.zeros((NF, C - 15), jnp.float32)],
        axis=1)

    lt = jnp.pad(_strided_rows(L_cols, 7, NV), padv)
    idxl = jnp.concatenate([lt, lt + NVP], axis=1).reshape(-1)
    wl = jnp.pad(_strided_rows(L_vals, 7, NV).T,
                 ((0, NVP - NV), (0, C - 7)))

    ft = jnp.pad(_strided_rows(F_cols, 6, NV), padv)
    idxf = jnp.concatenate([ft, ft + NF], axis=1).reshape(-1)
    wf = jnp.pad(_strided_rows(F_vals, 6, NV).T,
                 ((0, NVP - NV), (0, C - 6)))

    ewns, lap = _k1(xflat, idxg, wg, idxl, wl)
    gv = _k2(ewns, idxf, wf)

    C4 = coeffs.reshape(C, 4, 128)
    C0 = C4[:, 0, :]
    C1 = C4[:, 1, :]
    C23 = jnp.concatenate([C4[:, 2, :], C4[:, 3, :]], axis=0)

    return _final_matmul(xT, lap.reshape(BS, NVP, C),
                         gv.reshape(BS, NVP, 2 * C), C0, C1, C23)


# speedup vs baseline: 84.8882x; 1.0761x over previous
import functools
import jax
import jax.numpy as jnp
from jax import lax
from jax.experimental import pallas as pl
from jax.experimental.pallas import tpu as pltpu
from jax.experimental.pallas import tpu_sc as plsc

NV = 40962
NV_PREV = 10242
NF = 81920
C = 128
BS = 2
NVP = 41472
TV = 512

NW = 32
CH = 32
TA = BS * NF // CH
TL = BS * NVP // CH
TF = BS * NVP // CH
NCA = NF // CH
NCV = NVP // CH


def _sc_mesh():
    return plsc.VectorSubcoreMesh(core_axis_name="c", subcore_axis_name="s")


def _run_stage(ntasks, wid, meta_idx, meta_w, wait_idx, wait_w, issue,
               drain, compute, drain_out):
    ntw = -(-ntasks // NW)
    npairs = -(-ntw // 2)

    meta_idx(0, wid)
    meta_w(0, wid)
    wait_idx(0)
    issue(0, wid)

    @pl.when(wid + NW < ntasks)
    def _():
        meta_idx(1, wid + NW)
        meta_w(1, wid + NW)

    def pair(i2, _):
        te = wid + (2 * i2) * NW
        to = te + NW
        te2 = to + NW
        to2 = te2 + NW

        @pl.when(te < ntasks)
        def _():
            drain(0)

            @pl.when(to < ntasks)
            def _():
                wait_idx(1)
                issue(1, to)

            @pl.when(te2 < ntasks)
            def _():
                meta_idx(0, te2)

            @pl.when(te >= wid + 2 * NW)
            def _():
                drain_out(0)

            wait_w(0)
            compute(0, te)

            @pl.when(te2 < ntasks)
            def _():
                meta_w(0, te2)

        @pl.when(to < ntasks)
        def _():
            drain(1)

            @pl.when(te2 < ntasks)
            def _():
                wait_idx(0)
                issue(0, te2)

            @pl.when(to2 < ntasks)
            def _():
                meta_idx(1, to2)

            @pl.when(to >= wid + 2 * NW)
            def _():
                drain_out(1)

            wait_w(1)
            compute(1, to)

            @pl.when(to2 < ntasks)
            def _():
                meta_w(1, to2)

        return 0

    lax.fori_loop(0, npairs, pair, 0)

    drain_out(0)
    @pl.when(wid + NW < ntasks)
    def _():
        drain_out(1)


def _k1(xT, idxg, gvals, ew, ns, idxl, lvals):
    @functools.partial(
        pl.kernel, mesh=_sc_mesh(),
        out_type=[
            jax.ShapeDtypeStruct((BS * NF, 2 * C), jnp.float32),
            jax.ShapeDtypeStruct((BS * NVP, C), jnp.float32),
        ],
        scratch_types=[
            pltpu.VMEM((2, 3, 96), jnp.int32),
            pltpu.VMEM((2, 2, 112), jnp.int32),
            pltpu.VMEM((1024,), jnp.float32),
            pltpu.VMEM((256,), jnp.float32),
            pltpu.VMEM((256,), jnp.float32),
            pltpu.VMEM((512,), jnp.float32),
            pltpu.VMEM((2, 288, C), jnp.float32),
            pltpu.VMEM((2, CH, 2 * C), jnp.float32),
            pltpu.VMEM((2, CH, C), jnp.float32),
            pltpu.SemaphoreType.DMA,
            pltpu.SemaphoreType.DMA,
            pltpu.SemaphoreType.DMA,
            pltpu.SemaphoreType.DMA,
            pltpu.SemaphoreType.DMA,
            pltpu.SemaphoreType.DMA,
            pltpu.SemaphoreType.DMA,
            pltpu.SemaphoreType.DMA,
        ],
        compiler_params=pltpu.CompilerParams(needs_layout_passes=False),
    )
    def body(xT_h, idxg_h, gv_h, ew_h, ns_h, idxl_h, lv_h, ewns_h, lap_h,
             idxA2, idxL2, gw2, ew2, ns2, lw2, rows2, outA2, outL2,
             sg0, sg1, si0, si1, sw0, sw1, so0, so1):
        wid = lax.axis_index("s") * 2 + lax.axis_index("c")
        sg = [sg0, sg1]
        si = [si0, si1]
        sw = [sw0, sw1]
        so = [so0, so1]

        def metaA_idx(p, t):
            b = lax.div(t, NCA)
            off = b * (9 * NF) + (t - b * NCA) * (3 * CH)
            for d in range(3):
                pltpu.async_copy(idxg_h.at[pl.ds(off + d * 3 * NF, 3 * CH)],
                                 idxA2.at[p, d], si[p])

        def wait_idxA(p):
            for d in range(3):
                pltpu.make_async_copy(idxg_h.at[pl.ds(0, 3 * CH)],
                                      idxA2.at[p, d], si[p]).wait()

        def metaA_w(p, t):
            cf = lax.rem(t, NCA)
            for d in range(3):
                pltpu.async_copy(gv_h.at[pl.ds(cf * 3 * CH + d * 3 * NF,
                                               128)],
                                 gw2.at[pl.ds(p * 512 + d * 128, 128)],
                                 sw[p])
            pltpu.async_copy(ew_h.at[pl.ds(cf * 3 * CH, 128)],
                             ew2.at[pl.ds(p * 128, 128)], sw[p])
            pltpu.async_copy(ns_h.at[pl.ds(cf * 3 * CH, 128)],
                             ns2.at[pl.ds(p * 128, 128)], sw[p])

        def wait_wA(p):
            for d in range(3):
                pltpu.make_async_copy(gv_h.at[pl.ds(0, 128)],
                                      gw2.at[pl.ds(p * 512 + d * 128, 128)],
                                      sw[p]).wait()
            pltpu.make_async_copy(ew_h.at[pl.ds(0, 128)],
                                  ew2.at[pl.ds(p * 128, 128)], sw[p]).wait()
            pltpu.make_async_copy(ns_h.at[pl.ds(0, 128)],
                                  ns2.at[pl.ds(p * 128, 128)], sw[p]).wait()

        def issueA(p, t):
            for d in range(3):
                pltpu.async_copy(xT_h.at[idxA2.at[p, d]],
                                 rows2.at[p, pl.ds(d * 3 * CH, 3 * CH)],
                                 sg[p])

        def drainA(p):
            for d in range(3):
                pltpu.make_async_copy(
                    xT_h.at[idxA2.at[p, d]],
                    rows2.at[p, pl.ds(d * 3 * CH, 3 * CH)], sg[p]).wait()

        def computeA(p, t):
            def face(f, _):
                f3 = 3 * f + lax.iota(jnp.int32, 16)
                gw = [plsc.load_gather(gw2, [p * 512 + d * 128 + f3])
                      for d in range(3)]
                ewv = plsc.load_gather(ew2, [p * 128 + f3])
                nsv = plsc.load_gather(ns2, [p * 128 + f3])
                for c0 in range(0, C, 16):
                    v = [[rows2[p, d * 3 * CH + 3 * f + k, pl.ds(c0, 16)]
                          for k in range(3)] for d in range(3)]
                    pd = [v[d][0] * gw[d][0] + v[d][1] * gw[d][1]
                          + v[d][2] * gw[d][2] for d in range(3)]
                    outA2[p, f, pl.ds(c0, 16)] = (
                        pd[0] * ewv[0] + pd[1] * ewv[1] + pd[2] * ewv[2])
                    outA2[p, f, pl.ds(C + c0, 16)] = (
                        pd[0] * nsv[0] + pd[1] * nsv[1] + pd[2] * nsv[2])
                return 0

            lax.fori_loop(0, CH, face, 0, unroll=2)
            pltpu.async_copy(outA2.at[p], ewns_h.at[pl.ds(t * CH, CH)], so[p])

        def drain_outA(p):
            pltpu.make_async_copy(outA2.at[p], ewns_h.at[pl.ds(0, CH)],
                                  so[p]).wait()

        _run_stage(TA, wid, metaA_idx, metaA_w, wait_idxA, wait_wA,
                   issueA, drainA, computeA, drain_outA)

        def metaL_idx(p, t):
            for h in range(2):
                pltpu.async_copy(idxl_h.at[pl.ds(t * 7 * CH + h * 112, 112)],
                                 idxL2.at[p, h], si[p])

        def wait_idxL(p):
            for h in range(2):
                pltpu.make_async_copy(idxl_h.at[pl.ds(0, 112)],
                                      idxL2.at[p, h], si[p]).wait()

        def metaL_w(p, t):
            cf = lax.rem(t, NCV)
            pltpu.async_copy(lv_h.at[pl.ds(cf * 7 * CH, 256)],
                             lw2.at[pl.ds(p * 256, 256)], sw[p])

        def wait_wL(p):
            pltpu.make_async_copy(lv_h.at[pl.ds(0, 256)],
                                  lw2.at[pl.ds(p * 256, 256)], sw[p]).wait()

        def issueL(p, t):
            for h in range(2):
                pltpu.async_copy(xT_h.at[idxL2.at[p, h]],
                                 rows2.at[p, pl.ds(h * 112, 112)], sg[p])

        def drainL(p):
            for h in range(2):
                pltpu.make_async_copy(
                    xT_h.at[idxL2.at[p, h]],
                    rows2.at[p, pl.ds(h * 112, 112)], sg[p]).wait()

        def computeL(p, t):
            def vert(f, _):
                wv = plsc.load_gather(
                    lw2, [p * 256 + 7 * f + lax.iota(jnp.int32, 16)])
                for c0 in range(0, C, 16):
                    v = [rows2[p, 7 * f + j, pl.ds(c0, 16)]
                         for j in range(7)]
                    acc = v[0] * wv[0]
                    for j in range(1, 7):
                        acc = acc + v[j] * wv[j]
                    outL2[p, f, pl.ds(c0, 16)] = acc
                return 0

            lax.fori_loop(0, CH, vert, 0, unroll=2)
            pltpu.async_copy(outL2.at[p], lap_h.at[pl.ds(t * CH, CH)], so[p])

        def drain_outL(p):
            pltpu.make_async_copy(outL2.at[p], lap_h.at[pl.ds(0, CH)],
                                  so[p]).wait()

        _run_stage(TL, wid, metaL_idx, metaL_w, wait_idxL, wait_wL,
                   issueL, drainL, computeL, drain_outL)

    return body(xT, idxg, gvals, ew, ns, idxl, lvals)


def _k2(ewns, idxf, fvals):
    @functools.partial(
        pl.kernel, mesh=_sc_mesh(),
        out_type=jax.ShapeDtypeStruct((BS * NVP, 2 * C), jnp.float32),
        scratch_types=[
            pltpu.VMEM((2, 2, 96), jnp.int32),
            pltpu.VMEM((512,), jnp.float32),
            pltpu.VMEM((2, 192, 2 * C), jnp.float32),
            pltpu.VMEM((2, CH, 2 * C), jnp.float32),
            pltpu.SemaphoreType.DMA,
            pltpu.SemaphoreType.DMA,
            pltpu.SemaphoreType.DMA,
            pltpu.SemaphoreType.DMA,
            pltpu.SemaphoreType.DMA,
            pltpu.SemaphoreType.DMA,
            pltpu.SemaphoreType.DMA,
            pltpu.SemaphoreType.DMA,
        ],
        compiler_params=pltpu.CompilerParams(needs_layout_passes=False),
    )
    def body(ewns_h, idxf_h, fv_h, gv_h, idx2, fw2, rows2, out2,
             sg0, sg1, si0, si1, sw0, sw1, so0, so1):
        wid = lax.axis_index("s") * 2 + lax.axis_index("c")
        sg = [sg0, sg1]
        si = [si0, si1]
        sw = [sw0, sw1]
        so = [so0, so1]

        def metaF_idx(p, t):
            for h in range(2):
                pltpu.async_copy(idxf_h.at[pl.ds(t * 6 * CH + h * 96, 96)],
                                 idx2.at[p, h], si[p])

        def wait_idxF(p):
            for h in range(2):
                pltpu.make_async_copy(idxf_h.at[pl.ds(0, 96)],
                                      idx2.at[p, h], si[p]).wait()

        def metaF_w(p, t):
            cf = lax.rem(t, NCV)
            pltpu.async_copy(fv_h.at[pl.ds(cf * 6 * CH, 256)],
                             fw2.at[pl.ds(p * 256, 256)], sw[p])

        def wait_wF(p):
            pltpu.make_async_copy(fv_h.at[pl.ds(0, 256)],
                                  fw2.at[pl.ds(p * 256, 256)], sw[p]).wait()

        def issueF(p, t):
            for h in range(2):
                pltpu.async_copy(ewns_h.at[idx2.at[p, h]],
                                 rows2.at[p, pl.ds(h * 96, 96)], sg[p])

        def drainF(p):
            for h in range(2):
                pltpu.make_async_copy(ewns_h.at[idx2.at[p, h]],
                                      rows2.at[p, pl.ds(h * 96, 96)],
                                      sg[p]).wait()

        def computeF(p, t):
            def vert(f, _):
                wv = plsc.load_gather(
                    fw2, [p * 256 + 6 * f + lax.iota(jnp.int32, 16)])
                for c0 in range(0, 2 * C, 16):
                    v = [rows2[p, 6 * f + j, pl.ds(c0, 16)]
                         for j in range(6)]
                    acc = v[0] * wv[0]
                    for j in range(1, 6):
                        acc = acc + v[j] * wv[j]
                    out2[p, f, pl.ds(c0, 16)] = acc
                return 0

            lax.fori_loop(0, CH, vert, 0, unroll=2)
            pltpu.async_copy(out2.at[p], gv_h.at[pl.ds(t * CH, CH)], so[p])

        def drain_outF(p):
            pltpu.make_async_copy(out2.at[p], gv_h.at[pl.ds(0, CH)],
                                  so[p]).wait()

        _run_stage(TF, wid, metaF_idx, metaF_w, wait_idxF, wait_wF,
                   issueF, drainF, computeF, drain_outF)

    return body(ewns, idxf, fvals)


def _final_matmul(xT, lap, gv, C0, C1, C23):
    def body(x_ref, l_ref, g_ref, c0_ref, c1_ref, c23_ref, o_ref):
        a = lax.dot_general(c0_ref[...], x_ref[0],
                            (((0,), (1,)), ((), ())),
                            preferred_element_type=jnp.float32)
        b = lax.dot_general(c1_ref[...], l_ref[0],
                            (((0,), (1,)), ((), ())),
                            preferred_element_type=jnp.float32)
        c = lax.dot_general(c23_ref[...], g_ref[0],
                            (((0,), (1,)), ((), ())),
                            preferred_element_type=jnp.float32)
        o_ref[0] = a + b + c

    return pl.pallas_call(
        body,
        grid=(BS, NVP // TV),
        in_specs=[
            pl.BlockSpec((1, TV, 128), lambda b, i: (b, i, 0)),
            pl.BlockSpec((1, TV, 128), lambda b, i: (b, i, 0)),
            pl.BlockSpec((1, TV, 256), lambda b, i: (b, i, 0)),
            pl.BlockSpec((128, 128), lambda b, i: (0, 0)),
            pl.BlockSpec((128, 128), lambda b, i: (0, 0)),
            pl.BlockSpec((256, 128), lambda b, i: (0, 0)),
        ],
        out_specs=pl.BlockSpec((1, 128, TV), lambda b, i: (b, 0, i)),
        out_shape=jax.ShapeDtypeStruct((BS, 128, NV), jnp.float32),
    )(xT, lap, gv, C0, C1, C23)


def kernel(input, coeffs, G_rows, G_cols, G_vals, L_rows, L_cols, L_vals,
           F_rows, F_cols, F_vals, NS, EW):
    xT = jnp.concatenate(
        [input.transpose(0, 2, 1),
         jnp.ones((BS, NVP - NV_PREV, C), jnp.float32)], axis=1)
    xflat = xT.reshape(BS * NVP, C)

    idxg = jnp.concatenate([G_cols, G_cols + NVP])
    gvp = jnp.pad(G_vals, (0, 256))
    lcp = jnp.pad(L_cols, (0, (NVP - NV) * 7))
    idxl = jnp.concatenate([lcp, lcp + NVP])
    lvp = jnp.pad(L_vals, (0, (NVP - NV) * 7 + 256))
    fcp = jnp.pad(F_cols, (0, (NVP - NV) * 6))
    idxf = jnp.concatenate([fcp, fcp + NF])
    fvp = jnp.pad(F_vals, (0, (NVP - NV) * 6 + 256))
    ewf = jnp.pad(EW.reshape(-1), (0, 256))
    nsf = jnp.pad(NS.reshape(-1), (0, 256))

    ewns, lap = _k1(xflat, idxg, gvp, ewf, nsf, idxl, lvp)
    gv = _k2(ewns, idxf, fvp)

    C4 = coeffs.reshape(C, 4, 128)
    C0 = C4[:, 0, :]
    C1 = C4[:, 1, :]
    C23 = jnp.concatenate([C4[:, 2, :], C4[:, 3, :]], axis=0)

    return _final_matmul(xT, lap.reshape(BS, NVP, C),
                         gv.reshape(BS, NVP, 2 * C), C0, C1, C23)
